# baseline (speedup 1.0000x reference)
import functools
import math

import jax
import jax.numpy as jnp
from jax import lax
from jax.experimental import pallas as pl
from jax.experimental.pallas import tpu as pltpu
from jax.experimental.pallas import tpu_sc as plsc

D_MODEL = 64
SCALE = math.sqrt(D_MODEL)

NUM_CORES = 2
NUM_SUBCORES = 16
NUM_WORKERS = NUM_CORES * NUM_SUBCORES

CHUNK = 512
STREAM = 128
LANES = 16


def _make_sc_gather(batch: int):
    assert batch % (NUM_WORKERS * CHUNK) == 0
    b_per_w = batch // NUM_WORKERS
    n_chunks = b_per_w // CHUNK

    mesh = plsc.VectorSubcoreMesh(core_axis_name="c", subcore_axis_name="s")

    @functools.partial(
        pl.kernel,
        mesh=mesh,
        out_type=jax.ShapeDtypeStruct((batch, D_MODEL), jnp.float32),
        scratch_types=[
            pltpu.VMEM((CHUNK,), jnp.int32),
            pltpu.VMEM((CHUNK, D_MODEL), jnp.float32),
            pltpu.SemaphoreType.DMA,
        ],
        compiler_params=pltpu.CompilerParams(use_tc_tiling_on_sc=False),
    )
    def sc_gather(ids_hbm, table_hbm, out_hbm, idx_v, rows_v, sem):
        wid = lax.axis_index("s") * NUM_CORES + lax.axis_index("c")
        base = wid * b_per_w

        def chunk_body(g, carry):
            off = base + g * CHUNK
            pltpu.sync_copy(ids_hbm.at[pl.ds(off, CHUNK)], idx_v)
            copies = []
            for k in range(CHUNK // STREAM):
                copies.append(pltpu.async_copy(
                    table_hbm.at[idx_v.at[pl.ds(k * STREAM, STREAM)]],
                    rows_v.at[pl.ds(k * STREAM, STREAM)],
                    sem,
                ))
            for c in copies:
                c.wait()

            def scale_body(i, c):
                for j in range(D_MODEL // LANES):
                    sl = pl.ds(j * LANES, LANES)
                    rows_v[i, sl] = rows_v[i, sl] * SCALE
                return c

            lax.fori_loop(0, CHUNK, scale_body, 0, unroll=4)
            pltpu.sync_copy(rows_v, out_hbm.at[pl.ds(off, CHUNK)])
            return carry

        lax.fori_loop(0, n_chunks, chunk_body, 0)

    return sc_gather


def kernel(input_ids, table):
    orig_shape = input_ids.shape
    flat_ids = input_ids.reshape(-1).astype(jnp.int32)
    out = _make_sc_gather(flat_ids.shape[0])(flat_ids, table)
    return out.reshape(*orig_shape, D_MODEL)


# speedup vs baseline: 1.0925x; 1.0925x over previous
import functools
import math

import jax
import jax.numpy as jnp
from jax import lax
from jax.experimental import pallas as pl
from jax.experimental.pallas import tpu as pltpu
from jax.experimental.pallas import tpu_sc as plsc

D_MODEL = 64
SCALE = math.sqrt(D_MODEL)

NUM_CORES = 2
NUM_SUBCORES = 16
NUM_WORKERS = NUM_CORES * NUM_SUBCORES

CHUNK = 256
STREAM = 128
NBUF = 4
AHEAD = NBUF - 2
LANES = 16


def _make_sc_gather(batch: int):
    assert batch % (NUM_WORKERS * CHUNK * NBUF) == 0
    b_per_w = batch // NUM_WORKERS
    n_chunks = b_per_w // CHUNK

    mesh = plsc.VectorSubcoreMesh(core_axis_name="c", subcore_axis_name="s")

    @functools.partial(
        pl.kernel,
        mesh=mesh,
        out_type=jax.ShapeDtypeStruct((batch, D_MODEL), jnp.float32),
        scratch_types=[
            pltpu.VMEM((b_per_w,), jnp.int32),
            [pltpu.VMEM((CHUNK, D_MODEL), jnp.float32) for _ in range(NBUF)],
            [pltpu.SemaphoreType.DMA for _ in range(NBUF)],
            [pltpu.SemaphoreType.DMA for _ in range(NBUF)],
        ],
        compiler_params=pltpu.CompilerParams(use_tc_tiling_on_sc=False),
    )
    def sc_gather(ids_hbm, table_hbm, out_hbm, ids_v, bufs, gsems, osems):
        wid = lax.axis_index("s") * NUM_CORES + lax.axis_index("c")
        base = wid * b_per_w
        pltpu.sync_copy(ids_hbm.at[pl.ds(base, b_per_w)], ids_v)

        def issue_gather(g, b):
            for k in range(CHUNK // STREAM):
                pltpu.async_copy(
                    table_hbm.at[ids_v.at[pl.ds(g * CHUNK + k * STREAM, STREAM)]],
                    bufs[b].at[pl.ds(k * STREAM, STREAM)],
                    gsems[b],
                )

        def drain_gather(g, b):
            pltpu.make_async_copy(
                out_hbm.at[pl.ds(base + g * CHUNK, CHUNK)], bufs[b], gsems[b]
            ).wait()

        def drain_out(g, b):
            pltpu.make_async_copy(
                bufs[b], out_hbm.at[pl.ds(base + g * CHUNK, CHUNK)], osems[b]
            ).wait()

        for b in range(AHEAD):
            issue_gather(b, b)

        def step(g, b):
            ng = g + AHEAD
            nb = (b + AHEAD) % NBUF

            @pl.when(ng < n_chunks)
            def _():
                @pl.when(ng >= NBUF)
                def _():
                    drain_out(ng - NBUF, nb)

                issue_gather(ng, nb)

            drain_gather(g, b)

            def scale_body(i, c):
                for j in range(D_MODEL // LANES):
                    sl = pl.ds(j * LANES, LANES)
                    bufs[b][i, sl] = bufs[b][i, sl] * SCALE
                return c

            lax.fori_loop(0, CHUNK, scale_body, 0, unroll=8)
            pltpu.async_copy(
                bufs[b], out_hbm.at[pl.ds(base + g * CHUNK, CHUNK)], osems[b]
            )

        def outer(t, carry):
            for b in range(NBUF):
                step(t * NBUF + b, b)
            return carry

        lax.fori_loop(0, n_chunks // NBUF, outer, 0)

        for b in range(NBUF):
            drain_out(n_chunks - NBUF + b, b)

    return sc_gather


def kernel(input_ids, table):
    orig_shape = input_ids.shape
    flat_ids = input_ids.reshape(-1).astype(jnp.int32)
    out = _make_sc_gather(flat_ids.shape[0])(flat_ids, table)
    return out.reshape(*orig_shape, D_MODEL)
